# baseline (speedup 1.0000x reference)
import jax
import jax.numpy as jnp
from jax import lax
from jax.experimental import pallas as pl
from jax.experimental.pallas import tpu as pltpu
from jax.experimental.pallas import tpu_sc as plsc

_NUM_CLASSES = 100000
_FEAT = 64
_BATCH = 16384
_NC = 2
_NS = 16
_L = 16
_SPB = _BATCH // _NS
_NIDX = _SPB // 128
_Q = 256
_NQ = _SPB // _Q
_CPQ = _Q // 128
_CHUNK = 16


def _body(feats_hbm, labels_hbm, out_hbm,
          labels_v, fv0, fv1, rq0, rq1, zrows_v, z128_v, ones_v,
          cg_v, acc_v,
          sem_l0, sem_l1, sem_s0, sem_s1, sem_g0, sem_g1,
          sums_sh, counts_sh):
    c = lax.axis_index("c")
    s = lax.axis_index("s")
    base = s * _SPB
    fv = (fv0, fv1)
    rq = (rq0, rq1)
    sem_l = (sem_l0, sem_l1)
    sem_s = (sem_s0, sem_s1)
    sem_g = (sem_g0, sem_g1)

    pltpu.sync_copy(labels_hbm.at[s], labels_v)

    zero16 = jnp.zeros((_L,), jnp.float32)
    one16 = jnp.ones((_L,), jnp.float32)

    def _fill0(i, _):
        zrows_v[i, :] = zero16
        return 0
    lax.fori_loop(0, 128, _fill0, 0)
    for j in range(128 // _L):
        z128_v[pl.ds(j * _L, _L)] = zero16
        ones_v[pl.ds(j * _L, _L)] = one16

    acc = jnp.zeros((_L,), jnp.float32)

    def _slab(col0, q):
        return feats_hbm.at[pl.ds(base + q * _Q, _Q), pl.ds(col0, _CHUNK)]

    for r in range(2):
        col0 = (c * 2 + r) * _CHUNK

        with jax.named_scope("phaseZ"):
            lds = [pltpu.async_copy(_slab(col0, 0), fv0, sem_l0),
                   pltpu.async_copy(_slab(col0, 1), fv1, sem_l1)]
            zds = []
            for j in range(_NIDX):
                idx = labels_v.at[j]
                zds.append(pltpu.async_copy(zrows_v, sums_sh.at[idx],
                                            sem_s0))
                if r == 0:
                    zds.append(pltpu.async_copy(z128_v, counts_sh.at[idx],
                                                sem_s1))
            for d in zds:
                d.wait()
            plsc.subcore_barrier()

        sds = [None, None]
        for q in range(_NQ):
          with jax.named_scope("phaseA"):
            b = q % 2
            ob = 1 - b
            lds[b].wait()
            sds[b] = []
            for j in range(_CPQ):
                idx = labels_v.at[q * _CPQ + j]
                sds[b].append(pltpu.async_copy(
                    fv[b].at[pl.ds(j * 128, 128)], sums_sh.at[idx],
                    sem_s[b], add=True))
            if sds[ob] is not None:
                for d in sds[ob]:
                    d.wait()
                sds[ob] = None
                if q + 1 < _NQ:
                    lds[ob] = pltpu.async_copy(_slab(col0, q + 1), fv[ob],
                                               sem_l[ob])
        with jax.named_scope("phaseA2"):
            cds = []
            if r == 0:
                for j in range(_NIDX):
                    cds.append(pltpu.async_copy(ones_v,
                                                counts_sh.at[labels_v.at[j]],
                                                sem_g0, add=True))
            for b in range(2):
                if sds[b] is not None:
                    for d in sds[b]:
                        d.wait()
            for d in cds:
                d.wait()
            plsc.subcore_barrier()

        if r == 0:
          with jax.named_scope("phaseC"):
            gds = []
            for j in range(_NIDX):
                gds.append(pltpu.async_copy(counts_sh.at[labels_v.at[j]],
                                            cg_v.at[pl.ds(j * 128, 128)],
                                            sem_g0))
            for d in gds:
                d.wait()

            def _inv(bk, _):
                cv = cg_v[pl.ds(bk * _L, _L)]
                iv = 1.0 / cv
                cg_v[pl.ds(bk * _L, _L)] = iv * iv
                return 0
            lax.fori_loop(0, _SPB // _L, _inv, 0)

        def _fire_g(q):
            b = q % 2
            ds_ = [pltpu.async_copy(_slab(col0, q), fv[b], sem_l[b])]
            for j in range(_CPQ):
                idx = labels_v.at[q * _CPQ + j]
                ds_.append(pltpu.async_copy(sums_sh.at[idx],
                                           rq[b].at[pl.ds(j * 128, 128)],
                                           sem_g[b]))
            return ds_

        gq = [None, None]
        gq[0] = _fire_g(0)
        for q in range(_NQ):
            b = q % 2
            with jax.named_scope("gwait"):
                if q + 1 < _NQ:
                    gq[(q + 1) % 2] = _fire_g(q + 1)
                for d in gq[b]:
                    d.wait()

            qoff = q * _Q

            def _accum(i, a):
                i2 = i * 2
                f0 = fv[b][i2, :]
                s0 = rq[b][i2, :]
                isq0 = cg_v[pl.ds(qoff + i2, _L)][0]
                a = a + (f0 * f0 - s0 * s0 * isq0)
                f1 = fv[b][i2 + 1, :]
                s1 = rq[b][i2 + 1, :]
                isq1 = cg_v[pl.ds(qoff + i2 + 1, _L)][0]
                return a + (f1 * f1 - s1 * s1 * isq1)
            with jax.named_scope("compute"):
                acc = lax.fori_loop(0, _Q // 2, _accum, acc)
        if r == 0:
            plsc.subcore_barrier()

    acc_v[...] = acc * (0.5 / _BATCH)
    pltpu.sync_copy(acc_v, out_hbm.at[c, s])


@jax.jit
def kernel(feats, labels, centers):
    del centers
    labels_r = labels.astype(jnp.int32).reshape(_NS, _NIDX, 128)
    mesh = plsc.VectorSubcoreMesh(core_axis_name="c", subcore_axis_name="s")
    partials = pl.kernel(
        _body,
        out_type=jax.ShapeDtypeStruct((_NC, _NS, _L), jnp.float32),
        mesh=mesh,
        compiler_params=pltpu.CompilerParams(use_tc_tiling_on_sc=False),
        scratch_types=[
            pltpu.VMEM((_NIDX, 128), jnp.int32),
            pltpu.VMEM((_Q, _CHUNK), jnp.float32),
            pltpu.VMEM((_Q, _CHUNK), jnp.float32),
            pltpu.VMEM((_Q, _CHUNK), jnp.float32),
            pltpu.VMEM((_Q, _CHUNK), jnp.float32),
            pltpu.VMEM((128, _CHUNK), jnp.float32),
            pltpu.VMEM((128,), jnp.float32),
            pltpu.VMEM((128,), jnp.float32),
            pltpu.VMEM((_SPB + _L,), jnp.float32),
            pltpu.VMEM((_L,), jnp.float32),
            pltpu.SemaphoreType.DMA,
            pltpu.SemaphoreType.DMA,
            pltpu.SemaphoreType.DMA,
            pltpu.SemaphoreType.DMA,
            pltpu.SemaphoreType.DMA,
            pltpu.SemaphoreType.DMA,
            pltpu.VMEM_SHARED((_NUM_CLASSES, _CHUNK), jnp.float32),
            pltpu.VMEM_SHARED((_NUM_CLASSES,), jnp.float32),
        ],
    )(feats, labels_r)
    return jnp.sum(partials)


# speedup vs baseline: 1.0548x; 1.0548x over previous
import jax
import jax.numpy as jnp
from jax import lax
from jax.experimental import pallas as pl
from jax.experimental.pallas import tpu as pltpu
from jax.experimental.pallas import tpu_sc as plsc

_NUM_CLASSES = 100000
_FEAT = 64
_BATCH = 16384
_NC = 2
_NS = 16
_L = 16
_SPB = _BATCH // _NS
_NIDX = _SPB // 128
_Q = 256
_NQ = _SPB // _Q
_CPQ = _Q // 128
_CHUNK = 16


def _body(feats_hbm, labels_hbm, out_hbm,
          labels_v, q0, q1, q2, q3, zrows_v, z128_v, ones_v,
          cg_v, acc_v,
          sem_q0, sem_q1, sem_q2, sem_q3, sem_s, sem_c,
          sums_sh, counts_sh):
    c = lax.axis_index("c")
    s = lax.axis_index("s")
    base = s * _SPB
    qb = (q0, q1, q2, q3)
    sem_q = (sem_q0, sem_q1, sem_q2, sem_q3)

    pltpu.sync_copy(labels_hbm.at[s], labels_v)

    zero16 = jnp.zeros((_L,), jnp.float32)
    one16 = jnp.ones((_L,), jnp.float32)

    def _fill0(i, _):
        zrows_v[i, :] = zero16
        return 0
    lax.fori_loop(0, 128, _fill0, 0)
    for j in range(128 // _L):
        z128_v[pl.ds(j * _L, _L)] = zero16
        ones_v[pl.ds(j * _L, _L)] = one16

    acc = jnp.zeros((_L,), jnp.float32)

    def _slab(col0, q):
        return feats_hbm.at[pl.ds(base + q * _Q, _Q), pl.ds(col0, _CHUNK)]

    for r in range(2):
        col0 = (c * 2 + r) * _CHUNK

        with jax.named_scope("phZ"):
            lds = [pltpu.async_copy(_slab(col0, q), qb[q], sem_q[q])
                   for q in range(_NQ)]
            zds = []
            for j in range(_NIDX):
                idx = labels_v.at[j]
                zds.append(pltpu.async_copy(zrows_v, sums_sh.at[idx],
                                            sem_s))
                if r == 0:
                    zds.append(pltpu.async_copy(z128_v, counts_sh.at[idx],
                                                sem_c))
            for d in zds:
                d.wait()
            plsc.subcore_barrier()

        with jax.named_scope("phAfire"):
            sds = []
            for q in range(_NQ):
                lds[q].wait()
                for j in range(_CPQ):
                    idx = labels_v.at[q * _CPQ + j]
                    sds.append(pltpu.async_copy(
                        qb[q].at[pl.ds(j * 128, 128)], sums_sh.at[idx],
                        sem_s, add=True))
            if r == 0:
                for j in range(_NIDX):
                    sds.append(pltpu.async_copy(
                        ones_v, counts_sh.at[labels_v.at[j]],
                        sem_c, add=True))

        with jax.named_scope("phAff"):
            for q in range(_NQ):
                def _ff(i, a):
                    i2 = i * 2
                    f0 = qb[q][i2, :]
                    f1 = qb[q][i2 + 1, :]
                    return a + f0 * f0 + f1 * f1
                acc = lax.fori_loop(0, _Q // 2, _ff, acc)

        with jax.named_scope("phAdrain"):
            for d in sds:
                d.wait()
            plsc.subcore_barrier()

        gds = [[] for _ in range(_NQ)]
        with jax.named_scope("phGfire"):
            for q in range(_NQ):
                for j in range(_CPQ):
                    idx = labels_v.at[q * _CPQ + j]
                    gds[q].append(pltpu.async_copy(
                        sums_sh.at[idx], qb[q].at[pl.ds(j * 128, 128)],
                        sem_q[q]))
        if r == 0:
          with jax.named_scope("phC"):
            cds = []
            for j in range(_NIDX):
                cds.append(pltpu.async_copy(counts_sh.at[labels_v.at[j]],
                                            cg_v.at[pl.ds(j * 128, 128)],
                                            sem_c))
            for d in cds:
                d.wait()

            def _inv(bk, _):
                cv = cg_v[pl.ds(bk * _L, _L)]
                iv = 1.0 / cv
                cg_v[pl.ds(bk * _L, _L)] = iv * iv
                return 0
            lax.fori_loop(0, _SPB // _L, _inv, 0)

        for q in range(_NQ):
            with jax.named_scope("phGwait"):
                for d in gds[q]:
                    d.wait()
            qoff = q * _Q

            with jax.named_scope("phGsg"):
                def _sg(i, a):
                    i2 = i * 2
                    s0 = qb[q][i2, :]
                    isq0 = cg_v[pl.ds(qoff + i2, _L)][0]
                    a = a - s0 * s0 * isq0
                    s1 = qb[q][i2 + 1, :]
                    isq1 = cg_v[pl.ds(qoff + i2 + 1, _L)][0]
                    return a - s1 * s1 * isq1
                acc = lax.fori_loop(0, _Q // 2, _sg, acc)
        if r == 0:
            plsc.subcore_barrier()

    acc_v[...] = acc * (0.5 / _BATCH)
    pltpu.sync_copy(acc_v, out_hbm.at[c, s])


@jax.jit
def kernel(feats, labels, centers):
    del centers
    labels_r = labels.astype(jnp.int32).reshape(_NS, _NIDX, 128)
    mesh = plsc.VectorSubcoreMesh(core_axis_name="c", subcore_axis_name="s")
    partials = pl.kernel(
        _body,
        out_type=jax.ShapeDtypeStruct((_NC, _NS, _L), jnp.float32),
        mesh=mesh,
        compiler_params=pltpu.CompilerParams(use_tc_tiling_on_sc=False),
        scratch_types=[
            pltpu.VMEM((_NIDX, 128), jnp.int32),
            pltpu.VMEM((_Q, _CHUNK), jnp.float32),
            pltpu.VMEM((_Q, _CHUNK), jnp.float32),
            pltpu.VMEM((_Q, _CHUNK), jnp.float32),
            pltpu.VMEM((_Q, _CHUNK), jnp.float32),
            pltpu.VMEM((128, _CHUNK), jnp.float32),
            pltpu.VMEM((128,), jnp.float32),
            pltpu.VMEM((128,), jnp.float32),
            pltpu.VMEM((_SPB + _L,), jnp.float32),
            pltpu.VMEM((_L,), jnp.float32),
            pltpu.SemaphoreType.DMA,
            pltpu.SemaphoreType.DMA,
            pltpu.SemaphoreType.DMA,
            pltpu.SemaphoreType.DMA,
            pltpu.SemaphoreType.DMA,
            pltpu.SemaphoreType.DMA,
            pltpu.VMEM_SHARED((_NUM_CLASSES, _CHUNK), jnp.float32),
            pltpu.VMEM_SHARED((_NUM_CLASSES,), jnp.float32),
        ],
    )(feats, labels_r)
    return jnp.sum(partials)
